# baseline (speedup 1.0000x reference)
import functools

import jax
import jax.numpy as jnp
from jax.experimental import pallas as pl
from jax.experimental.pallas import tpu as pltpu

_ROT = ((13, 15, 26, 6), (17, 29, 16, 24))
_K1 = 0
_K2 = 42
_K3 = _K1 ^ _K2 ^ 0x1BD11BDA


def _rotl(v, d):
    return (v << jnp.uint32(d)) | (v >> jnp.uint32(32 - d))


def _threefry_bits(i):
    ks = (jnp.uint32(_K1), jnp.uint32(_K2), jnp.uint32(_K3))
    x0 = jnp.full_like(i, ks[0])
    x1 = i + ks[1]
    for rnd in range(5):
        for r in _ROT[rnd % 2]:
            x0 = x0 + x1
            x1 = _rotl(x1, r)
            x1 = x0 ^ x1
        x0 = x0 + ks[(rnd + 1) % 3]
        x1 = x1 + ks[(rnd + 2) % 3] + jnp.uint32(rnd + 1)
    return x0 ^ x1


_P_CENTRAL = (2.81022636e-08, 3.43273939e-07, -3.5233877e-06, -4.39150654e-06,
              0.00021858087, -0.00125372503, -0.00417768164, 0.246640727,
              1.50140941)
_P_TAIL = (-0.000200214257, 0.000100950558, 0.00134934322, -0.00367342844,
           0.00573950773, -0.0076224613, 0.00943887047, 1.00167406,
           2.83297682)


def _erfinv(x):
    w = -jnp.log1p(-x * x)
    wl = w - jnp.float32(2.5)
    p1 = jnp.float32(_P_CENTRAL[0])
    for c in _P_CENTRAL[1:]:
        p1 = jnp.float32(c) + p1 * wl
    wg = jnp.sqrt(w) - jnp.float32(3.0)
    p2 = jnp.float32(_P_TAIL[0])
    for c in _P_TAIL[1:]:
        p2 = jnp.float32(c) + p2 * wg
    return jnp.where(w < jnp.float32(5.0), p1, p2) * x


def _noise_kernel(o_ref, *, rows_per_block, ncols):
    r0 = pl.program_id(0) * rows_per_block
    shape = (rows_per_block, ncols)
    row = jax.lax.broadcasted_iota(jnp.uint32, shape, 0)
    col = jax.lax.broadcasted_iota(jnp.uint32, shape, 1)
    i = (jnp.uint32(r0) + row) * jnp.uint32(ncols) + col
    bits = _threefry_bits(i)
    mant = (bits >> jnp.uint32(9)) | jnp.uint32(0x3F800000)
    f = jax.lax.bitcast_convert_type(mant, jnp.float32) - jnp.float32(1.0)
    lo = jnp.float32(-0.99999994)
    hi = jnp.float32(1.0)
    u = jnp.maximum(lo, f * (hi - lo) + lo)
    o_ref[...] = jnp.float32(1.4142135623730951) * _erfinv(u)


@functools.partial(jax.jit, static_argnames=())
def kernel(x):
    nrows, ncols = x.shape
    rows_per_block = 8
    grid = (nrows // rows_per_block,)
    out = pl.pallas_call(
        functools.partial(_noise_kernel, rows_per_block=rows_per_block,
                          ncols=ncols),
        grid=grid,
        out_specs=pl.BlockSpec((rows_per_block, ncols), lambda b: (b, 0)),
        out_shape=jax.ShapeDtypeStruct((nrows, ncols), jnp.float32),
        compiler_params=pltpu.CompilerParams(
            dimension_semantics=("parallel",),
        ),
    )()
    return out.astype(x.dtype)
